# Initial kernel scaffold; baseline (speedup 1.0000x reference)
import functools

import jax
import jax.numpy as jnp
from jax import lax
from jax.experimental import pallas as pl
from jax.experimental.pallas import tpu as pltpu
from jax.experimental.pallas import tpu_sc as plsc

NC = 2
NS = 16
L = 16

_MESH = plsc.VectorSubcoreMesh(
    core_axis_name="c", subcore_axis_name="s", num_cores=NC, num_subcores=NS
)


def _sc_degrees(ei, n):
    e = ei.shape[2]
    e_per = e // NS
    chunk = 800
    assert e_per % chunk == 0 and chunk % 8 == 0
    nchunks = e_per // chunk

    @functools.partial(
        pl.kernel,
        out_type=jax.ShapeDtypeStruct((2, 2, n), jnp.float32),
        mesh=_MESH,
        scratch_types=[
            pltpu.VMEM((chunk,), jnp.int32),
            pltpu.VMEM((chunk,), jnp.float32),
            pltpu.VMEM((n,), jnp.float32),
            pltpu.VMEM_SHARED((n,), jnp.float32),
            pltpu.VMEM_SHARED((n,), jnp.float32),
        ],
    )
    def deg_kernel(ei_hbm, degs_hbm, idx_v, ones_v, zeros_v, deg_sh0, deg_sh1):
        c = lax.axis_index("c")
        s = lax.axis_index("s")

        def fill_ones(i, _):
            ones_v[pl.ds(i * L, L)] = jnp.ones((L,), jnp.float32)
            return 0

        lax.fori_loop(0, chunk // L, fill_ones, 0)

        @pl.when(s == 0)
        def _():
            def fill_zeros(i, _):
                zeros_v[pl.ds(i * L, L)] = jnp.zeros((L,), jnp.float32)
                return 0

            lax.fori_loop(0, n // L, fill_zeros, 0)
            pltpu.sync_copy(zeros_v, deg_sh0)
            pltpu.sync_copy(zeros_v, deg_sh1)

        plsc.subcore_barrier()

        base = s * e_per

        def body(i, _):
            off = pl.multiple_of(base + i * chunk, 8)
            pltpu.sync_copy(ei_hbm.at[c, 0, pl.ds(off, chunk)], idx_v)
            pltpu.sync_copy(ones_v, deg_sh0.at[idx_v], add=True)
            pltpu.sync_copy(ei_hbm.at[c, 1, pl.ds(off, chunk)], idx_v)
            pltpu.sync_copy(ones_v, deg_sh1.at[idx_v], add=True)
            return 0

        lax.fori_loop(0, nchunks, body, 0)
        plsc.subcore_barrier()

        @pl.when(s == 0)
        def _():
            pltpu.sync_copy(deg_sh0, degs_hbm.at[c, 0])
            pltpu.sync_copy(deg_sh1, degs_hbm.at[c, 1])

    return deg_kernel(ei)


def _sc_aggregate(xs, ei):
    n, d = xs.shape[1], xs.shape[2]
    e = ei.shape[2]
    e_per = e // NS
    chunk = 800
    assert e_per % chunk == 0
    nchunks = e_per // chunk
    rpt = n // NS

    @functools.partial(
        pl.kernel,
        out_type=jax.ShapeDtypeStruct((2, n, d), jnp.float32),
        mesh=_MESH,
        scratch_types=[
            pltpu.VMEM((chunk,), jnp.int32),
            pltpu.VMEM((chunk,), jnp.int32),
            pltpu.VMEM((chunk, d), jnp.float32),
            pltpu.VMEM_SHARED((n, d), jnp.float32),
            pltpu.SemaphoreType.DMA,
        ],
    )
    def agg_kernel(xs_hbm, ei_hbm, agg_hbm, src_v, dst_v, rows_v, agg_sh, sem):
        c = lax.axis_index("c")
        s = lax.axis_index("s")

        def fill_zero(i, _):
            rows_v[i // (d // L), pl.ds((i % (d // L)) * L, L)] = jnp.zeros(
                (L,), jnp.float32
            )
            return 0

        lax.fori_loop(0, rpt * (d // L), fill_zero, 0)
        pltpu.sync_copy(
            rows_v.at[pl.ds(0, rpt)], agg_sh.at[pl.ds(s * rpt, rpt)]
        )
        plsc.subcore_barrier()

        base = s * e_per

        def body(i, _):
            off = pl.multiple_of(base + i * chunk, 8)
            pltpu.sync_copy(ei_hbm.at[c, 0, pl.ds(off, chunk)], src_v)
            pltpu.sync_copy(ei_hbm.at[c, 1, pl.ds(off, chunk)], dst_v)
            pltpu.async_copy(xs_hbm.at[c].at[src_v], rows_v, sem).wait()
            pltpu.sync_copy(rows_v, agg_sh.at[dst_v], add=True)
            return 0

        lax.fori_loop(0, nchunks, body, 0)
        plsc.subcore_barrier()

        pltpu.sync_copy(
            agg_sh.at[pl.ds(s * rpt, rpt)],
            agg_hbm.at[c, pl.ds(s * rpt, rpt)],
        )

    return agg_kernel(xs, ei)


def _tc_scale_matmul(h, degs, wstack, bn):
    n, din = h.shape
    dout = wstack.shape[2]
    nb = n // bn

    def body(h_ref, degs_ref, w_ref, xs_ref):
        c = pl.program_id(0)
        i = pl.program_id(1)
        deg = degs_ref[c, 0, pl.ds(i * bn, bn)]
        f = lax.rsqrt(jnp.maximum(deg, 1.0))
        xh = h_ref[...] * f[:, None]
        xs_ref[0] = jnp.dot(xh, w_ref[0], preferred_element_type=jnp.float32)

    return pl.pallas_call(
        body,
        grid=(2, nb),
        in_specs=[
            pl.BlockSpec((bn, din), lambda c, i: (i, 0)),
            pl.BlockSpec((2, 2, n), lambda c, i: (0, 0, 0)),
            pl.BlockSpec((1, din, dout), lambda c, i: (c, 0, 0)),
        ],
        out_specs=pl.BlockSpec((1, bn, dout), lambda c, i: (c, i, 0)),
        out_shape=jax.ShapeDtypeStruct((2, n, dout), jnp.float32),
    )(h, degs, wstack)


def _tc_emb_scores(agg, degs, bstack, p1, pb1, p2, bn):
    _, n, d = agg.shape
    hid = p1.shape[1]
    nb = n // bn

    def body(agg_ref, degs_ref, b_ref, p1_ref, pb1_ref, p2_ref, emb_ref, sc_ref):
        c = pl.program_id(0)
        i = pl.program_id(1)
        deg = degs_ref[c, 1, pl.ds(i * bn, bn)]
        f = lax.rsqrt(jnp.maximum(deg, 1.0))
        emb = agg_ref[0] * f[:, None] + b_ref[0]
        emb_ref[0] = emb
        t = jnp.tanh(
            jnp.dot(emb, p1_ref[...], preferred_element_type=jnp.float32)
            + pb1_ref[...]
        )
        sc = jnp.dot(t, p2_ref[...], preferred_element_type=jnp.float32)
        part = jnp.sum(sc)

        @pl.when(i == 0)
        def _():
            sc_ref[...] = jnp.zeros_like(sc_ref)

        sc_ref[...] += part

    return pl.pallas_call(
        body,
        grid=(2, nb),
        in_specs=[
            pl.BlockSpec((1, bn, d), lambda c, i: (c, i, 0)),
            pl.BlockSpec((2, 2, n), lambda c, i: (0, 0, 0)),
            pl.BlockSpec((1, 1, d), lambda c, i: (c, 0, 0)),
            pl.BlockSpec((d, hid), lambda c, i: (0, 0)),
            pl.BlockSpec((1, hid), lambda c, i: (0, 0)),
            pl.BlockSpec((hid, 1), lambda c, i: (0, 0)),
        ],
        out_specs=[
            pl.BlockSpec((1, bn, d), lambda c, i: (c, i, 0)),
            pl.BlockSpec((8, 128), lambda c, i: (c, 0)),
        ],
        out_shape=[
            jax.ShapeDtypeStruct((2, n, d), jnp.float32),
            jax.ShapeDtypeStruct((16, 128), jnp.float32),
        ],
    )(agg, degs, bstack, p1, pb1, p2)


def _tc_combine(emb, scores, bn):
    _, n, d = emb.shape
    nb = n // bn
    inv_n = 1.0 / n

    def body(emb_ref, sc_ref, out_ref):
        s0 = sc_ref[0:1, 0:1]
        s1 = sc_ref[8:9, 0:1]
        m = jnp.maximum(s0, s1)
        e0 = jnp.exp((s0 - m) * inv_n)
        e1 = jnp.exp((s1 - m) * inv_n)
        tot = e0 + e1
        out_ref[...] = emb_ref[0] * (e0 / tot) + emb_ref[1] * (e1 / tot)

    return pl.pallas_call(
        body,
        grid=(nb,),
        in_specs=[
            pl.BlockSpec((2, bn, d), lambda i: (0, i, 0)),
            pl.BlockSpec((16, 128), lambda i: (0, 0)),
        ],
        out_specs=pl.BlockSpec((bn, d), lambda i: (i, 0)),
        out_shape=jax.ShapeDtypeStruct((n, d), jnp.float32),
    )(emb, scores)


def kernel(h, edge_index_0, edge_index_1, W0, b0, W1, b1, P1, pb1, P2):
    n = h.shape[0]
    ei = jnp.stack([edge_index_0, edge_index_1])
    wstack = jnp.stack([W0, W1])
    bstack = jnp.stack([b0, b1]).reshape(2, 1, -1)

    degs = _sc_degrees(ei, n)
    xs = _tc_scale_matmul(h, degs, wstack, bn=1000)
    agg = _sc_aggregate(xs, ei)
    emb, scores = _tc_emb_scores(
        agg, degs, bstack, P1, pb1.reshape(1, -1), P2, bn=1000
    )
    return _tc_combine(emb, scores, bn=1000)


# speedup vs baseline: 8.1016x; 8.1016x over previous
import functools

import jax
import jax.numpy as jnp
from jax import lax
from jax.experimental import pallas as pl
from jax.experimental.pallas import tpu as pltpu
from jax.experimental.pallas import tpu_sc as plsc

NC = 2
NS = 16
L = 16

_MESH = plsc.VectorSubcoreMesh(
    core_axis_name="c", subcore_axis_name="s", num_cores=NC, num_subcores=NS
)


def _sc_degrees(eif, e, n):
    e_per = e // NS
    chunk = 800
    assert e_per % chunk == 0 and chunk % 8 == 0
    nchunks = e_per // chunk

    @functools.partial(
        pl.kernel,
        out_type=jax.ShapeDtypeStruct((4 * n,), jnp.float32),
        mesh=_MESH,
        scratch_types=[
            pltpu.VMEM((chunk,), jnp.int32),
            pltpu.VMEM((chunk,), jnp.float32),
            pltpu.VMEM((n,), jnp.float32),
            pltpu.VMEM_SHARED((n,), jnp.float32),
            pltpu.VMEM_SHARED((n,), jnp.float32),
        ],
    )
    def deg_kernel(ei_hbm, degs_hbm, idx_v, ones_v, zeros_v, deg_sh0, deg_sh1):
        c = lax.axis_index("c")
        s = lax.axis_index("s")

        def fill_ones(i, _):
            ones_v[pl.ds(i * L, L)] = jnp.ones((L,), jnp.float32)
            return 0

        lax.fori_loop(0, chunk // L, fill_ones, 0)

        @pl.when(s == 0)
        def _():
            def fill_zeros(i, _):
                zeros_v[pl.ds(i * L, L)] = jnp.zeros((L,), jnp.float32)
                return 0

            lax.fori_loop(0, n // L, fill_zeros, 0)
            pltpu.sync_copy(zeros_v, deg_sh0)
            pltpu.sync_copy(zeros_v, deg_sh1)

        plsc.subcore_barrier()

        src_base = 2 * c * e + s * e_per
        dst_base = (2 * c + 1) * e + s * e_per

        def body(i, _):
            pltpu.sync_copy(
                ei_hbm.at[pl.ds(pl.multiple_of(src_base + i * chunk, 8), chunk)],
                idx_v,
            )
            pltpu.sync_copy(ones_v, deg_sh0.at[idx_v], add=True)
            pltpu.sync_copy(
                ei_hbm.at[pl.ds(pl.multiple_of(dst_base + i * chunk, 8), chunk)],
                idx_v,
            )
            pltpu.sync_copy(ones_v, deg_sh1.at[idx_v], add=True)
            return 0

        lax.fori_loop(0, nchunks, body, 0)
        plsc.subcore_barrier()

        @pl.when(s == 0)
        def _():
            pltpu.sync_copy(deg_sh0, zeros_v)
            pltpu.sync_copy(
                zeros_v, degs_hbm.at[pl.ds(pl.multiple_of(2 * c * n, 8), n)]
            )
            pltpu.sync_copy(deg_sh1, zeros_v)
            pltpu.sync_copy(
                zeros_v, degs_hbm.at[pl.ds(pl.multiple_of((2 * c + 1) * n, 8), n)]
            )

    return deg_kernel(eif)


def _sc_aggregate(xs, eif, e):
    n, d = xs.shape[1], xs.shape[2]
    e_per = e // NS
    chunk = 800
    assert e_per % chunk == 0
    nchunks = e_per // chunk
    nfull = n // chunk
    tail = n - nfull * chunk
    assert tail % 8 == 0

    @functools.partial(
        pl.kernel,
        out_type=jax.ShapeDtypeStruct((2, n, d), jnp.float32),
        mesh=_MESH,
        scratch_types=[
            pltpu.VMEM((chunk,), jnp.int32),
            pltpu.VMEM((chunk,), jnp.int32),
            pltpu.VMEM((chunk, d), jnp.float32),
            pltpu.VMEM_SHARED((n, d), jnp.float32),
            pltpu.SemaphoreType.DMA,
        ],
        compiler_params=pltpu.CompilerParams(use_tc_tiling_on_sc=False),
    )
    def agg_kernel(xs_hbm, ei_hbm, agg_hbm, src_v, dst_v, rows_v, agg_sh, sem):
        c = lax.axis_index("c")
        s = lax.axis_index("s")

        def fill_zero(i, _):
            rows_v[i // (d // L), pl.ds((i % (d // L)) * L, L)] = jnp.zeros(
                (L,), jnp.float32
            )
            return 0

        lax.fori_loop(0, chunk * (d // L), fill_zero, 0)

        @pl.when(s < nfull)
        def _():
            pltpu.sync_copy(
                rows_v, agg_sh.at[pl.ds(pl.multiple_of(s * chunk, 8), chunk)]
            )

        @pl.when(s == nfull)
        def _():
            pltpu.sync_copy(
                rows_v.at[pl.ds(0, tail)],
                agg_sh.at[pl.ds(nfull * chunk, tail)],
            )

        plsc.subcore_barrier()

        src_base = 2 * c * e + s * e_per
        dst_base = (2 * c + 1) * e + s * e_per

        def body(i, _):
            pltpu.sync_copy(
                ei_hbm.at[pl.ds(pl.multiple_of(src_base + i * chunk, 8), chunk)],
                src_v,
            )
            pltpu.sync_copy(
                ei_hbm.at[pl.ds(pl.multiple_of(dst_base + i * chunk, 8), chunk)],
                dst_v,
            )
            pltpu.async_copy(xs_hbm.at[c].at[src_v], rows_v, sem).wait()
            pltpu.sync_copy(rows_v, agg_sh.at[dst_v], add=True)
            return 0

        lax.fori_loop(0, nchunks, body, 0)
        plsc.subcore_barrier()

        @pl.when(s < nfull)
        def _():
            off = pl.multiple_of(s * chunk, 8)
            pltpu.sync_copy(agg_sh.at[pl.ds(off, chunk)], rows_v)
            pltpu.sync_copy(rows_v, agg_hbm.at[c, pl.ds(off, chunk)])

        @pl.when(s == nfull)
        def _():
            pltpu.sync_copy(
                agg_sh.at[pl.ds(nfull * chunk, tail)], rows_v.at[pl.ds(0, tail)]
            )
            pltpu.sync_copy(
                rows_v.at[pl.ds(0, tail)],
                agg_hbm.at[c, pl.ds(nfull * chunk, tail)],
            )

    return agg_kernel(xs, eif)


def _sel_column(f2, c):
    onehot = (lax.broadcasted_iota(jnp.int32, (2, 1), 0) == c).astype(jnp.float32)
    return jnp.dot(f2, onehot, preferred_element_type=jnp.float32)


def _tc_scale_matmul(h, out_t, wstack, bn):
    n, din = h.shape
    dout = wstack.shape[2]
    nb = n // bn

    def body(h_ref, deg_ref, w_ref, xs_ref):
        c = pl.program_id(0)
        f2 = lax.rsqrt(jnp.maximum(deg_ref[...], 1.0))
        f = _sel_column(f2, c)
        xm = jnp.dot(h_ref[...], w_ref[0], preferred_element_type=jnp.float32)
        xs_ref[0] = xm * f

    return pl.pallas_call(
        body,
        grid=(2, nb),
        in_specs=[
            pl.BlockSpec((bn, din), lambda c, i: (i, 0)),
            pl.BlockSpec((bn, 2), lambda c, i: (i, 0)),
            pl.BlockSpec((1, din, dout), lambda c, i: (c, 0, 0)),
        ],
        out_specs=pl.BlockSpec((1, bn, dout), lambda c, i: (c, i, 0)),
        out_shape=jax.ShapeDtypeStruct((2, n, dout), jnp.float32),
    )(h, out_t, wstack)


def _tc_emb_scores(agg, in_t, bstack, p1, pb1, p2, bn):
    _, n, d = agg.shape
    hid = p1.shape[1]
    nb = n // bn

    def body(agg_ref, deg_ref, b_ref, p1_ref, pb1_ref, p2_ref, emb_ref, sc_ref):
        c = pl.program_id(0)
        i = pl.program_id(1)
        f2 = lax.rsqrt(jnp.maximum(deg_ref[...], 1.0))
        f = _sel_column(f2, c)
        emb = agg_ref[0] * f + b_ref[0]
        emb_ref[0] = emb
        t = jnp.tanh(
            jnp.dot(emb, p1_ref[...], preferred_element_type=jnp.float32)
            + pb1_ref[...]
        )
        sc = jnp.dot(t, p2_ref[...], preferred_element_type=jnp.float32)
        part = jnp.sum(sc)

        @pl.when(i == 0)
        def _():
            sc_ref[...] = jnp.zeros_like(sc_ref)

        sc_ref[...] += part

    return pl.pallas_call(
        body,
        grid=(2, nb),
        in_specs=[
            pl.BlockSpec((1, bn, d), lambda c, i: (c, i, 0)),
            pl.BlockSpec((bn, 2), lambda c, i: (i, 0)),
            pl.BlockSpec((1, 1, d), lambda c, i: (c, 0, 0)),
            pl.BlockSpec((d, hid), lambda c, i: (0, 0)),
            pl.BlockSpec((1, hid), lambda c, i: (0, 0)),
            pl.BlockSpec((hid, 1), lambda c, i: (0, 0)),
        ],
        out_specs=[
            pl.BlockSpec((1, bn, d), lambda c, i: (c, i, 0)),
            pl.BlockSpec((8, 128), lambda c, i: (c, 0)),
        ],
        out_shape=[
            jax.ShapeDtypeStruct((2, n, d), jnp.float32),
            jax.ShapeDtypeStruct((16, 128), jnp.float32),
        ],
    )(agg, in_t, bstack, p1, pb1, p2)


def _tc_combine(emb, scores, bn):
    _, n, d = emb.shape
    nb = n // bn
    inv_n = 1.0 / n

    def body(emb_ref, sc_ref, out_ref):
        s0 = sc_ref[0:1, 0:1]
        s1 = sc_ref[8:9, 0:1]
        m = jnp.maximum(s0, s1)
        e0 = jnp.exp((s0 - m) * inv_n)
        e1 = jnp.exp((s1 - m) * inv_n)
        tot = e0 + e1
        out_ref[...] = emb_ref[0] * (e0 / tot) + emb_ref[1] * (e1 / tot)

    return pl.pallas_call(
        body,
        grid=(nb,),
        in_specs=[
            pl.BlockSpec((2, bn, d), lambda i: (0, i, 0)),
            pl.BlockSpec((16, 128), lambda i: (0, 0)),
        ],
        out_specs=pl.BlockSpec((bn, d), lambda i: (i, 0)),
        out_shape=jax.ShapeDtypeStruct((n, d), jnp.float32),
    )(emb, scores)


def kernel(h, edge_index_0, edge_index_1, W0, b0, W1, b1, P1, pb1, P2):
    n = h.shape[0]
    e = edge_index_0.shape[1]
    eif = jnp.concatenate(
        [edge_index_0.reshape(-1), edge_index_1.reshape(-1)]
    )
    wstack = jnp.stack([W0, W1])
    bstack = jnp.stack([b0, b1]).reshape(2, 1, -1)

    degs = _sc_degrees(eif, e, n).reshape(2, 2, n)
    out_t = jnp.transpose(degs[:, 0, :])
    in_t = jnp.transpose(degs[:, 1, :])
    xs = _tc_scale_matmul(h, out_t, wstack, bn=1000)
    agg = _sc_aggregate(xs, eif, e)
    emb, scores = _tc_emb_scores(
        agg, in_t, bstack, P1, pb1.reshape(1, -1), P2, bn=1000
    )
    return _tc_combine(emb, scores, bn=1000)


# speedup vs baseline: 9.8073x; 1.2105x over previous
import functools

import jax
import jax.numpy as jnp
from jax import lax
from jax.experimental import pallas as pl
from jax.experimental.pallas import tpu as pltpu
from jax.experimental.pallas import tpu_sc as plsc

NC = 2
NS = 16
L = 16

_MESH = plsc.VectorSubcoreMesh(
    core_axis_name="c", subcore_axis_name="s", num_cores=NC, num_subcores=NS
)


def _sc_degrees(eif, e, n):
    e_per = e // NS
    chunk = 800
    assert e_per % chunk == 0 and chunk % 8 == 0
    nchunks = e_per // chunk

    @functools.partial(
        pl.kernel,
        out_type=jax.ShapeDtypeStruct((4 * n,), jnp.float32),
        mesh=_MESH,
        scratch_types=[
            pltpu.VMEM((chunk,), jnp.int32),
            pltpu.VMEM((chunk,), jnp.float32),
            pltpu.VMEM((n,), jnp.float32),
            pltpu.VMEM_SHARED((n,), jnp.float32),
            pltpu.VMEM_SHARED((n,), jnp.float32),
        ],
    )
    def deg_kernel(ei_hbm, degs_hbm, idx_v, ones_v, zeros_v, deg_sh0, deg_sh1):
        c = lax.axis_index("c")
        s = lax.axis_index("s")

        def fill_ones(i, _):
            ones_v[pl.ds(i * L, L)] = jnp.ones((L,), jnp.float32)
            return 0

        lax.fori_loop(0, chunk // L, fill_ones, 0)

        @pl.when(s == 0)
        def _():
            def fill_zeros(i, _):
                zeros_v[pl.ds(i * L, L)] = jnp.zeros((L,), jnp.float32)
                return 0

            lax.fori_loop(0, n // L, fill_zeros, 0)
            pltpu.sync_copy(zeros_v, deg_sh0)
            pltpu.sync_copy(zeros_v, deg_sh1)

        plsc.subcore_barrier()

        src_base = 2 * c * e + s * e_per
        dst_base = (2 * c + 1) * e + s * e_per

        def body(i, _):
            pltpu.sync_copy(
                ei_hbm.at[pl.ds(pl.multiple_of(src_base + i * chunk, 8), chunk)],
                idx_v,
            )
            pltpu.sync_copy(ones_v, deg_sh0.at[idx_v], add=True)
            pltpu.sync_copy(
                ei_hbm.at[pl.ds(pl.multiple_of(dst_base + i * chunk, 8), chunk)],
                idx_v,
            )
            pltpu.sync_copy(ones_v, deg_sh1.at[idx_v], add=True)
            return 0

        lax.fori_loop(0, nchunks, body, 0)
        plsc.subcore_barrier()

        @pl.when(s == 0)
        def _():
            pltpu.sync_copy(deg_sh0, zeros_v)
            pltpu.sync_copy(
                zeros_v, degs_hbm.at[pl.ds(pl.multiple_of(2 * c * n, 8), n)]
            )
            pltpu.sync_copy(deg_sh1, zeros_v)
            pltpu.sync_copy(
                zeros_v, degs_hbm.at[pl.ds(pl.multiple_of((2 * c + 1) * n, 8), n)]
            )

    return deg_kernel(eif)


def _sc_aggregate(xs, eif, e):
    n, d = xs.shape[1], xs.shape[2]
    e_per = e // NS
    chunk = 400
    assert e_per % (2 * chunk) == 0 and chunk % 8 == 0
    nchunks = e_per // chunk
    assert n % chunk == 0
    nwb = n // chunk
    wrounds = (nwb + NS - 1) // NS

    @functools.partial(
        pl.kernel,
        out_type=jax.ShapeDtypeStruct((2, n, d), jnp.float32),
        mesh=_MESH,
        scratch_types=[
            pltpu.VMEM((chunk,), jnp.int32),
            pltpu.VMEM((chunk,), jnp.int32),
            pltpu.VMEM((chunk,), jnp.int32),
            pltpu.VMEM((chunk,), jnp.int32),
            pltpu.VMEM((chunk, d), jnp.float32),
            pltpu.VMEM((chunk, d), jnp.float32),
            pltpu.VMEM_SHARED((n, d), jnp.float32),
            pltpu.SemaphoreType.DMA,
            pltpu.SemaphoreType.DMA,
            pltpu.SemaphoreType.DMA,
            pltpu.SemaphoreType.DMA,
            pltpu.SemaphoreType.DMA,
            pltpu.SemaphoreType.DMA,
        ],
        compiler_params=pltpu.CompilerParams(use_tc_tiling_on_sc=False),
    )
    def agg_kernel(
        xs_hbm, ei_hbm, agg_hbm,
        src0, dst0, src1, dst1, rows0, rows1, agg_sh,
        si0, si1, sg0, sg1, ss0, ss1,
    ):
        c = lax.axis_index("c")
        s = lax.axis_index("s")
        src_v = (src0, src1)
        dst_v = (dst0, dst1)
        rows_v = (rows0, rows1)
        si = (si0, si1)
        sg = (sg0, sg1)
        ss = (ss0, ss1)

        def fill_zero(i, _):
            rows0[i // (d // L), pl.ds((i % (d // L)) * L, L)] = jnp.zeros(
                (L,), jnp.float32
            )
            return 0

        lax.fori_loop(0, chunk * (d // L), fill_zero, 0)

        for z in range(wrounds):
            blk = s + z * NS

            @pl.when(blk < nwb)
            def _(blk=blk):
                pltpu.sync_copy(
                    rows0, agg_sh.at[pl.ds(pl.multiple_of(blk * chunk, 8), chunk)]
                )

        plsc.subcore_barrier()

        src_base = 2 * c * e + s * e_per
        dst_base = (2 * c + 1) * e + s * e_per

        def start_idx(b, i):
            pltpu.async_copy(
                ei_hbm.at[pl.ds(pl.multiple_of(src_base + i * chunk, 8), chunk)],
                src_v[b], si[b],
            )
            pltpu.async_copy(
                ei_hbm.at[pl.ds(pl.multiple_of(dst_base + i * chunk, 8), chunk)],
                dst_v[b], si[b],
            )

        def wait_idx(b):
            pltpu.make_async_copy(
                ei_hbm.at[pl.ds(0, chunk)], src_v[b], si[b]
            ).wait()
            pltpu.make_async_copy(
                ei_hbm.at[pl.ds(0, chunk)], dst_v[b], si[b]
            ).wait()

        def start_gather(b):
            pltpu.async_copy(xs_hbm.at[c].at[src_v[b]], rows_v[b], sg[b])

        def wait_gather(b):
            pltpu.make_async_copy(
                xs_hbm.at[c].at[src_v[b]], rows_v[b], sg[b]
            ).wait()

        def start_scatter(b):
            pltpu.async_copy(rows_v[b], agg_sh.at[dst_v[b]], ss[b], add=True)

        def wait_scatter(b):
            pltpu.make_async_copy(rows_v[b], agg_sh.at[dst_v[b]], ss[b]).wait()

        start_idx(0, 0)
        start_idx(1, 1)
        wait_idx(0)
        start_gather(0)

        def body(k, _):
            i0 = 2 * k
            wait_gather(0)
            wait_idx(1)
            start_gather(1)
            start_scatter(0)
            wait_scatter(0)

            @pl.when(i0 + 2 < nchunks)
            def _():
                start_idx(0, i0 + 2)

            wait_gather(1)

            @pl.when(i0 + 2 < nchunks)
            def _():
                wait_idx(0)
                start_gather(0)

            start_scatter(1)
            wait_scatter(1)

            @pl.when(i0 + 3 < nchunks)
            def _():
                start_idx(1, i0 + 3)

            return 0

        lax.fori_loop(0, nchunks // 2, body, 0)
        plsc.subcore_barrier()

        for z in range(wrounds):
            blk = s + z * NS

            @pl.when(blk < nwb)
            def _(blk=blk):
                off = pl.multiple_of(blk * chunk, 8)
                pltpu.sync_copy(agg_sh.at[pl.ds(off, chunk)], rows0)
                pltpu.sync_copy(rows0, agg_hbm.at[c, pl.ds(off, chunk)])

    return agg_kernel(xs, eif)


def _sel_column(f2, c):
    onehot = (lax.broadcasted_iota(jnp.int32, (2, 1), 0) == c).astype(jnp.float32)
    return jnp.dot(f2, onehot, preferred_element_type=jnp.float32)


def _tc_scale_matmul(h, out_t, wstack, bn):
    n, din = h.shape
    dout = wstack.shape[2]
    nb = n // bn

    def body(h_ref, deg_ref, w_ref, xs_ref):
        c = pl.program_id(0)
        f2 = lax.rsqrt(jnp.maximum(deg_ref[...], 1.0))
        f = _sel_column(f2, c)
        xm = jnp.dot(h_ref[...], w_ref[0], preferred_element_type=jnp.float32)
        xs_ref[0] = xm * f

    return pl.pallas_call(
        body,
        grid=(2, nb),
        in_specs=[
            pl.BlockSpec((bn, din), lambda c, i: (i, 0)),
            pl.BlockSpec((bn, 2), lambda c, i: (i, 0)),
            pl.BlockSpec((1, din, dout), lambda c, i: (c, 0, 0)),
        ],
        out_specs=pl.BlockSpec((1, bn, dout), lambda c, i: (c, i, 0)),
        out_shape=jax.ShapeDtypeStruct((2, n, dout), jnp.float32),
    )(h, out_t, wstack)


def _tc_emb_scores(agg, in_t, bstack, p1, pb1, p2, bn):
    _, n, d = agg.shape
    hid = p1.shape[1]
    nb = n // bn

    def body(agg_ref, deg_ref, b_ref, p1_ref, pb1_ref, p2_ref, emb_ref, sc_ref):
        c = pl.program_id(0)
        i = pl.program_id(1)
        f2 = lax.rsqrt(jnp.maximum(deg_ref[...], 1.0))
        f = _sel_column(f2, c)
        emb = agg_ref[0] * f + b_ref[0]
        emb_ref[0] = emb
        t = jnp.tanh(
            jnp.dot(emb, p1_ref[...], preferred_element_type=jnp.float32)
            + pb1_ref[...]
        )
        sc = jnp.dot(t, p2_ref[...], preferred_element_type=jnp.float32)
        part = jnp.sum(sc)

        @pl.when(i == 0)
        def _():
            sc_ref[...] = jnp.zeros_like(sc_ref)

        sc_ref[...] += part

    return pl.pallas_call(
        body,
        grid=(2, nb),
        in_specs=[
            pl.BlockSpec((1, bn, d), lambda c, i: (c, i, 0)),
            pl.BlockSpec((bn, 2), lambda c, i: (i, 0)),
            pl.BlockSpec((1, 1, d), lambda c, i: (c, 0, 0)),
            pl.BlockSpec((d, hid), lambda c, i: (0, 0)),
            pl.BlockSpec((1, hid), lambda c, i: (0, 0)),
            pl.BlockSpec((hid, 1), lambda c, i: (0, 0)),
        ],
        out_specs=[
            pl.BlockSpec((1, bn, d), lambda c, i: (c, i, 0)),
            pl.BlockSpec((8, 128), lambda c, i: (c, 0)),
        ],
        out_shape=[
            jax.ShapeDtypeStruct((2, n, d), jnp.float32),
            jax.ShapeDtypeStruct((16, 128), jnp.float32),
        ],
    )(agg, in_t, bstack, p1, pb1, p2)


def _tc_combine(emb, scores, bn):
    _, n, d = emb.shape
    nb = n // bn
    inv_n = 1.0 / n

    def body(emb_ref, sc_ref, out_ref):
        s0 = sc_ref[0:1, 0:1]
        s1 = sc_ref[8:9, 0:1]
        m = jnp.maximum(s0, s1)
        e0 = jnp.exp((s0 - m) * inv_n)
        e1 = jnp.exp((s1 - m) * inv_n)
        tot = e0 + e1
        out_ref[...] = emb_ref[0] * (e0 / tot) + emb_ref[1] * (e1 / tot)

    return pl.pallas_call(
        body,
        grid=(nb,),
        in_specs=[
            pl.BlockSpec((2, bn, d), lambda i: (0, i, 0)),
            pl.BlockSpec((16, 128), lambda i: (0, 0)),
        ],
        out_specs=pl.BlockSpec((bn, d), lambda i: (i, 0)),
        out_shape=jax.ShapeDtypeStruct((n, d), jnp.float32),
    )(emb, scores)


def kernel(h, edge_index_0, edge_index_1, W0, b0, W1, b1, P1, pb1, P2):
    n = h.shape[0]
    e = edge_index_0.shape[1]
    eif = jnp.concatenate(
        [edge_index_0.reshape(-1), edge_index_1.reshape(-1)]
    )
    wstack = jnp.stack([W0, W1])
    bstack = jnp.stack([b0, b1]).reshape(2, 1, -1)

    degs = _sc_degrees(eif, e, n).reshape(2, 2, n)
    out_t = jnp.transpose(degs[:, 0, :])
    in_t = jnp.transpose(degs[:, 1, :])
    xs = _tc_scale_matmul(h, out_t, wstack, bn=1000)
    agg = _sc_aggregate(xs, eif, e)
    emb, scores = _tc_emb_scores(
        agg, in_t, bstack, P1, pb1.reshape(1, -1), P2, bn=1000
    )
    return _tc_combine(emb, scores, bn=1000)
